# baseline (speedup 1.0000x reference)
import functools

import jax
import jax.numpy as jnp
from jax import lax
from jax.experimental import pallas as pl
from jax.experimental.pallas import tpu as pltpu
from jax.experimental.pallas import tpu_sc as plsc

_B = 16384
_D = 128
_C = 64
_NB = 4
_NC = 2
_NS = 16
_NW = _NC * _NS
_RPW = _B // _NW
_K = _RPW // _C


def _embed_body(charge_hbm, spin_hbm, dataset_hbm, ct_hbm, st_hbm, dt_hbm,
                bias_hbm, out_hbm, ct_sh, st_sh, dt_sh, idx_c, idx_s, idx_d,
                *rest):
    rows = [rest[3 * b:3 * b + 3] for b in range(_NB)]
    outs = [rest[3 * _NB], rest[3 * _NB + 1]]
    bias_v = rest[3 * _NB + 2]
    isem = rest[3 * _NB + 3]
    tsem = rest[3 * _NB + 4]
    gsems = rest[3 * _NB + 5:3 * _NB + 5 + _NB]
    osems = rest[3 * _NB + 5 + _NB:]

    sid = lax.axis_index("s")
    wid = sid * _NC + lax.axis_index("c")
    base = wid * _RPW

    cp_i = (pltpu.async_copy(charge_hbm.at[pl.ds(base, _RPW)], idx_c, isem),
            pltpu.async_copy(spin_hbm.at[pl.ds(base, _RPW)], idx_s, isem),
            pltpu.async_copy(dataset_hbm.at[pl.ds(base, _RPW)], idx_d, isem),
            pltpu.async_copy(bias_hbm, bias_v, isem))

    cp_t = ()

    @pl.when(sid == 0)
    def _load_tables():
        pltpu.async_copy(ct_hbm, ct_sh, tsem)
        pltpu.async_copy(st_hbm, st_sh, tsem)
        pltpu.async_copy(dt_hbm, dt_sh, tsem)

    for cp in cp_i:
        cp.wait()

    for i in range(_RPW // 16):
        sl = pl.ds(i * 16, 16)
        idx_c[sl] = idx_c[sl] + 100

    bias_regs = [bias_v[pl.ds(j * 16, 16)] for j in range(8)]

    @pl.when(sid == 0)
    def _wait_tables():
        pltpu.make_async_copy(ct_hbm, ct_sh, tsem).wait()
        pltpu.make_async_copy(st_hbm, st_sh, tsem).wait()
        pltpu.make_async_copy(dt_hbm, dt_sh, tsem).wait()

    plsc.subcore_barrier()

    def issue_gather(k):
        b = k % _NB
        rc, rs, rd = rows[b]
        sl = pl.ds(k * _C, _C)
        return (pltpu.async_copy(ct_sh.at[idx_c.at[sl]], rc, gsems[b]),
                pltpu.async_copy(st_sh.at[idx_s.at[sl]], rs, gsems[b]),
                pltpu.async_copy(dt_sh.at[idx_d.at[sl]], rd, gsems[b]))

    pending_g = [None] * _NB
    pending_out = [None, None]
    for k in range(_NB - 1):
        pending_g[k % _NB] = issue_gather(k)

    for k in range(_K):
        b = k % _NB
        ob = k % 2
        for cp in pending_g[b]:
            cp.wait()
        if pending_out[ob] is not None:
            pending_out[ob].wait()
        rc, rs, rd = rows[b]
        ov = outs[ob]

        def row_body(r, carry):
            for j in range(8):
                sl = pl.ds(j * 16, 16)
                x = rc[r, sl] + rs[r, sl] + rd[r, sl] + bias_regs[j]
                ov[r, sl] = x / (1.0 + jnp.exp(-x))
            return carry

        lax.fori_loop(0, _C, row_body, 0)

        if k + _NB - 1 < _K:
            pending_g[(k + _NB - 1) % _NB] = issue_gather(k + _NB - 1)

        pending_out[ob] = pltpu.async_copy(
            ov, out_hbm.at[pl.ds(base + k * _C, _C)], osems[ob])

    pending_out[0].wait()
    pending_out[1].wait()


@jax.jit
def _embed(charge, spin, dataset, charge_table, spin_table, dataset_table,
           bias):
    mesh = plsc.VectorSubcoreMesh(core_axis_name="c", subcore_axis_name="s")
    scratch = [
        pltpu.VMEM_SHARED((201, _D), jnp.float32),
        pltpu.VMEM_SHARED((101, _D), jnp.float32),
        pltpu.VMEM_SHARED((1000, _D), jnp.float32),
        pltpu.VMEM((_RPW,), jnp.int32),
        pltpu.VMEM((_RPW,), jnp.int32),
        pltpu.VMEM((_RPW,), jnp.int32),
    ]
    scratch += [pltpu.VMEM((_C, _D), jnp.float32) for _ in range(3 * _NB + 2)]
    scratch += [pltpu.VMEM((_D,), jnp.float32)]
    scratch += [pltpu.SemaphoreType.DMA for _ in range(2 + _NB + 2)]
    kern = pl.kernel(
        _embed_body,
        mesh=mesh,
        out_type=jax.ShapeDtypeStruct((_B, _D), jnp.float32),
        scratch_types=scratch,
    )
    return kern(charge, spin, dataset, charge_table, spin_table,
                dataset_table, bias)


def kernel(charge, spin, dataset, charge_table, spin_table, dataset_table, bias):
    return _embed(charge, spin, dataset, charge_table, spin_table,
                  dataset_table, bias)


# speedup vs baseline: 1.0020x; 1.0020x over previous
import jax
import jax.numpy as jnp
from jax import lax
from jax.experimental import pallas as pl
from jax.experimental.pallas import tpu as pltpu
from jax.experimental.pallas import tpu_sc as plsc

_B = 16384
_D = 128
_C = 64
_NB = 4
_NC = 2
_NS = 16
_NW = _NC * _NS
_RPW = _B // _NW
_K = _RPW // _C


def _embed_body(charge_hbm, spin_hbm, dataset_hbm, ct_hbm, st_hbm, dt_hbm,
                bias_hbm, out_hbm, ct_sh, st_sh, dt_sh, idx_c, idx_s, idx_d,
                *rest):
    rows = [rest[3 * b:3 * b + 3] for b in range(_NB)]
    outs = [rest[3 * _NB], rest[3 * _NB + 1]]
    bias_v = rest[3 * _NB + 2]
    isem = rest[3 * _NB + 3]
    tsem = rest[3 * _NB + 4]
    gsems = rest[3 * _NB + 5:3 * _NB + 5 + _NB]
    osems = rest[3 * _NB + 5 + _NB:]

    sid = lax.axis_index("s")
    wid = sid * _NC + lax.axis_index("c")
    base = wid * _RPW

    cp_i = (pltpu.async_copy(charge_hbm.at[pl.ds(base, _RPW)], idx_c, isem),
            pltpu.async_copy(spin_hbm.at[pl.ds(base, _RPW)], idx_s, isem),
            pltpu.async_copy(dataset_hbm.at[pl.ds(base, _RPW)], idx_d, isem),
            pltpu.async_copy(bias_hbm, bias_v, isem))

    @pl.when(sid == 0)
    def _load_tables():
        pltpu.async_copy(ct_hbm, ct_sh, tsem)
        pltpu.async_copy(st_hbm, st_sh, tsem)
        pltpu.async_copy(dt_hbm, dt_sh, tsem)

    for cp in cp_i:
        cp.wait()

    for i in range(_RPW // 16):
        sl = pl.ds(i * 16, 16)
        idx_c[sl] = idx_c[sl] + 100

    bias_regs = [bias_v[pl.ds(j * 16, 16)] for j in range(8)]

    @pl.when(sid == 0)
    def _wait_tables():
        pltpu.make_async_copy(ct_hbm, ct_sh, tsem).wait()
        pltpu.make_async_copy(st_hbm, st_sh, tsem).wait()
        pltpu.make_async_copy(dt_hbm, dt_sh, tsem).wait()

    plsc.subcore_barrier()

    def issue_gather(k):
        b = k % _NB
        rc, rs, rd = rows[b]
        sl = pl.ds(k * _C, _C)
        return (pltpu.async_copy(ct_sh.at[idx_c.at[sl]], rc, gsems[b]),
                pltpu.async_copy(st_sh.at[idx_s.at[sl]], rs, gsems[b]),
                pltpu.async_copy(dt_sh.at[idx_d.at[sl]], rd, gsems[b]))

    pending_g = [None] * _NB
    pending_out = [None, None]
    for k in range(_NB - 1):
        pending_g[k % _NB] = issue_gather(k)

    for k in range(_K):
        b = k % _NB
        ob = k % 2
        for cp in pending_g[b]:
            cp.wait()
        if pending_out[ob] is not None:
            pending_out[ob].wait()
        rc, rs, rd = rows[b]
        ov = outs[ob]

        def row_body(r, carry):
            for j in range(8):
                sl = pl.ds(j * 16, 16)
                x = rc[r, sl] + rs[r, sl] + rd[r, sl] + bias_regs[j]
                ov[r, sl] = x / (1.0 + jnp.exp(-x))
            return carry

        lax.fori_loop(0, _C, row_body, 0)

        if k + _NB - 1 < _K:
            pending_g[(k + _NB - 1) % _NB] = issue_gather(k + _NB - 1)

        pending_out[ob] = pltpu.async_copy(
            ov, out_hbm.at[pl.ds(base + k * _C, _C)], osems[ob])

    pending_out[0].wait()
    pending_out[1].wait()


@jax.jit
def _embed(charge, spin, dataset, charge_table, spin_table, dataset_table,
           bias):
    mesh = plsc.VectorSubcoreMesh(core_axis_name="c", subcore_axis_name="s")
    scratch = [
        pltpu.VMEM_SHARED((201, _D), jnp.float32),
        pltpu.VMEM_SHARED((101, _D), jnp.float32),
        pltpu.VMEM_SHARED((1000, _D), jnp.float32),
        pltpu.VMEM((_RPW,), jnp.int32),
        pltpu.VMEM((_RPW,), jnp.int32),
        pltpu.VMEM((_RPW,), jnp.int32),
    ]
    scratch += [pltpu.VMEM((_C, _D), jnp.float32) for _ in range(3 * _NB + 2)]
    scratch += [pltpu.VMEM((_D,), jnp.float32)]
    scratch += [pltpu.SemaphoreType.DMA for _ in range(2 + _NB + 2)]
    kern = pl.kernel(
        _embed_body,
        mesh=mesh,
        out_type=jax.ShapeDtypeStruct((_B, _D), jnp.float32),
        scratch_types=scratch,
    )
    return kern(charge, spin, dataset, charge_table, spin_table,
                dataset_table, bias)


def kernel(charge, spin, dataset, charge_table, spin_table, dataset_table, bias):
    return _embed(charge, spin, dataset, charge_table, spin_table,
                  dataset_table, bias)
